# baseline (speedup 1.0000x reference)
import functools

import jax
import jax.numpy as jnp
import numpy as np
from jax.experimental import pallas as pl

_RCR = 5.2
_RCA = 3.5
_NUM_SPECIES = 4
_NUM_PAIRS = 10
_ETA_R = 16.0
_ETA_A = 8.0
_A = 32
_NQ = 512
_NPAIR = _A * (_A - 1) // 2
_NL = _A * 16
_MB = 8
_RADIAL_F = _NUM_SPECIES * 16
_ANGULAR_F = _NUM_PAIRS * 4 * 8

_JQ, _KQ = np.triu_indices(_A, k=1)

_FC_COEF = (9.9999998695e-01, -2.4674003665e+00, 2.0293461123e+00,
            -6.6757576357e-01, 1.1751096555e-01, -1.2677815461e-02,
            7.9689343489e-04)


def _fc_poly(u):
    acc = np.float32(_FC_COEF[6])
    for c in _FC_COEF[5::-1]:
        acc = acc * u + np.float32(c)
    return jnp.where(u <= 1.0, acc, 0.0)


def _aev_body(species_ref, coords_ref, posj_ref, posk_ref, pidx_ref,
              dgj_ref, dgk_ref, posr_ref, shfr_ref, dgr_ref, out_ref):
    pi = np.float32(np.pi)

    sp = species_ref[:, 0, :]
    pos = coords_ref[:, :, :]

    posr = posr_ref[:, :, :]
    shfr = shfr_ref[0, :, :]
    dgr = dgr_ref[0, :, :]

    djr = pos[:, :, :, None] - posr[:, :, None, :]
    d2_r = jnp.sum(djr * djr, axis=1)
    fc_rf = _fc_poly(d2_r * np.float32(1.0 / (_RCR * _RCR)))
    fc_rf = fc_rf * (0.25 * (1.0 - dgr))
    d_r = jnp.sqrt(d2_r + dgr)
    rad_f = jnp.exp(-_ETA_R * (d_r - shfr) ** 2) * fc_rf
    sidx = jax.lax.broadcasted_iota(jnp.int32, (_MB, _A, _NUM_SPECIES), 2)
    oh = (sp[:, :, None] == sidx).astype(jnp.float32)
    rad_sf = jax.lax.dot_general(oh, rad_f, (((1,), (1,)), ((0,), (0,))),
                                 preferred_element_type=jnp.float32)
    radial = jnp.transpose(rad_sf.reshape(_MB, _NUM_SPECIES, _A, 16),
                           (0, 2, 1, 3)).reshape(_MB, _A, _RADIAL_F)

    posj = posj_ref[:, :, :]
    posk = posk_ref[:, :, :]
    diag_ij = dgj_ref[0, :, :][None]
    diag_ik = dgk_ref[0, :, :][None]

    dj = pos[:, :, :, None] - posj[:, :, None, :]
    d2_ij = jnp.sum(dj * dj, axis=1)
    dk = pos[:, :, :, None] - posk[:, :, None, :]
    d2_ik = jnp.sum(dk * dk, axis=1)
    ejk = posj - posk
    e2 = ejk * ejk
    d2_jk = (e2[:, 0:1, :] + e2[:, 1:2, :] + e2[:, 2:3, :])

    s2_ij = d2_ij + diag_ij
    s2_ik = d2_ik + diag_ik
    inv_ij = jax.lax.rsqrt(s2_ij)
    inv_ik = jax.lax.rsqrt(s2_ik)
    d_ij = s2_ij * inv_ij
    d_ik = s2_ik * inv_ik

    rca2inv = np.float32(1.0 / (_RCA * _RCA))
    fc_ij = _fc_poly(d2_ij * rca2inv) * (1.0 - diag_ij)
    fc_ik = _fc_poly(d2_ik * rca2inv) * (1.0 - diag_ik)
    w = 2.0 * fc_ij * fc_ik

    dotv = 0.5 * (d2_ij + d2_ik - d2_jk)
    cth = 0.95 * dotv * inv_ij * inv_ik
    sth = jnp.sqrt(jnp.maximum(1.0 - cth * cth, 0.0))
    dsum = jnp.minimum(0.5 * (d_ij + d_ik), 4.0)

    f2_0 = jnp.exp(-_ETA_A * (dsum - 0.9) ** 2)
    r = jnp.exp(10.4 * dsum)
    f2_1 = f2_0 * (r * np.float32(np.exp(-10.4 * 0.9 - 3.38)))
    f2_2 = f2_1 * (r * np.float32(np.exp(-10.4 * 1.55 - 3.38)))
    f2_3 = f2_2 * (r * np.float32(np.exp(-10.4 * 2.2 - 3.38)))

    shfz = ((pi / 16.0) + (pi / 8.0)
            * jax.lax.broadcasted_iota(jnp.int32, (1, 1, 8, 1), 2)
            .astype(jnp.float32))
    czh = 0.5 * jnp.cos(shfz)
    szh = 0.5 * jnp.sin(shfz)
    base = 0.5 + cth[:, :, None, :] * czh + sth[:, :, None, :] * szh
    f1 = base * base
    f1 = f1 * f1
    f1 = f1 * f1
    f1 = f1 * f1
    f1 = f1 * f1

    ang = jnp.concatenate(
        [(w * f2_0)[:, :, None, :] * f1, (w * f2_1)[:, :, None, :] * f1,
         (w * f2_2)[:, :, None, :] * f1, (w * f2_3)[:, :, None, :] * f1],
        axis=2)
    pidx = jnp.broadcast_to(pidx_ref[:, :, :], (_MB, _NUM_PAIRS, _NQ))
    pslot = jax.lax.broadcasted_iota(jnp.int32, (_MB, _NUM_PAIRS, _NQ), 1)
    p_oht = (pidx == pslot).astype(jnp.float32)

    ang_p = jax.lax.dot_general(ang, p_oht, (((3,), (2,)), ((0,), (0,))),
                                preferred_element_type=jnp.float32)
    fr = jax.lax.broadcasted_iota(jnp.int32, (_ANGULAR_F, _ANGULAR_F), 0)
    fcol = jax.lax.broadcasted_iota(jnp.int32, (_ANGULAR_F, _ANGULAR_F), 1)
    perm = ((fr % 10) * 32 + fr // 10 == fcol).astype(jnp.float32)
    angular = jax.lax.dot_general(
        ang_p.reshape(_MB, _A, _ANGULAR_F), perm, (((2,), (0,)), ((), ())),
        preferred_element_type=jnp.float32)

    out_ref[:, :, :] = jnp.concatenate([radial, angular], axis=2)


@jax.jit
def _aev_pallas(species, coordinates):
    M, A = species.shape
    sp3 = species.astype(jnp.int32).reshape(M, 1, A)
    coords_t = jnp.transpose(coordinates, (0, 2, 1))

    jq = jnp.asarray(_JQ, dtype=jnp.int32)
    kq = jnp.asarray(_KQ, dtype=jnp.int32)
    npad = _NQ - _NPAIR
    posj = jnp.concatenate(
        [jnp.take(coords_t, jq, axis=2),
         jnp.full((M, 3, npad), 1.0e4, jnp.float32)], axis=2)
    posk = jnp.concatenate(
        [jnp.take(coords_t, kq, axis=2),
         jnp.full((M, 3, npad), 2.0e4, jnp.float32)], axis=2)

    spi = species.astype(jnp.int32)
    spj = jnp.take(spi, jq, axis=1)
    spk = jnp.take(spi, kq, axis=1)
    mn = jnp.minimum(spj, spk)
    mx = jnp.maximum(spj, spk)
    pidx = (mn * (7 - mn)) // 2 + mx
    pidx = jnp.pad(pidx, ((0, 0), (0, npad))).reshape(M, 1, _NQ)

    jq_pad = np.pad(_JQ, (0, npad), constant_values=-1)
    kq_pad = np.pad(_KQ, (0, npad), constant_values=-1)
    irows = np.arange(A)[:, None]
    dgj = jnp.asarray((jq_pad[None, :] == irows).astype(np.float32)
                      ).reshape(1, A, _NQ)
    dgk = jnp.asarray((kq_pad[None, :] == irows).astype(np.float32)
                      ).reshape(1, A, _NQ)

    posr = jnp.repeat(coords_t, 16, axis=2)
    shfr_np = np.tile(0.9 + 0.26875 * np.arange(16, dtype=np.float32), A)
    shfr_flat = jnp.asarray(shfr_np).reshape(1, 1, _NL)
    dgr = jnp.asarray(
        ((np.arange(_NL) // 16)[None, :] == irows).astype(np.float32)
    ).reshape(1, A, _NL)

    out = pl.pallas_call(
        _aev_body,
        grid=(M // _MB,),
        in_specs=[
            pl.BlockSpec((_MB, 1, A), lambda m: (m, 0, 0)),
            pl.BlockSpec((_MB, 3, A), lambda m: (m, 0, 0)),
            pl.BlockSpec((_MB, 3, _NQ), lambda m: (m, 0, 0)),
            pl.BlockSpec((_MB, 3, _NQ), lambda m: (m, 0, 0)),
            pl.BlockSpec((_MB, 1, _NQ), lambda m: (m, 0, 0)),
            pl.BlockSpec((1, A, _NQ), lambda m: (0, 0, 0)),
            pl.BlockSpec((1, A, _NQ), lambda m: (0, 0, 0)),
            pl.BlockSpec((_MB, 3, _NL), lambda m: (m, 0, 0)),
            pl.BlockSpec((1, 1, _NL), lambda m: (0, 0, 0)),
            pl.BlockSpec((1, A, _NL), lambda m: (0, 0, 0)),
        ],
        out_specs=pl.BlockSpec((_MB, A, _RADIAL_F + _ANGULAR_F),
                               lambda m: (m, 0, 0)),
        out_shape=jax.ShapeDtypeStruct((M, A, _RADIAL_F + _ANGULAR_F),
                                       jnp.float32),
    )(sp3, coords_t, posj, posk, pidx, dgj, dgk, posr, shfr_flat, dgr)
    return out


def kernel(species, coordinates):
    aev = _aev_pallas(species, coordinates)
    return (species, aev)


# speedup vs baseline: 1.0265x; 1.0265x over previous
import functools

import jax
import jax.numpy as jnp
import numpy as np
from jax.experimental import pallas as pl

_RCR = 5.2
_RCA = 3.5
_NUM_SPECIES = 4
_NUM_PAIRS = 10
_ETA_R = 16.0
_ETA_A = 8.0
_A = 32
_NQ = 512
_NPAIR = _A * (_A - 1) // 2
_NL = _A * 16
_MB = 8
_RADIAL_F = _NUM_SPECIES * 16
_ANGULAR_F = _NUM_PAIRS * 4 * 8

_JQ, _KQ = np.triu_indices(_A, k=1)

_FC_COEF = (9.9999998695e-01, -2.4674003665e+00, 2.0293461123e+00,
            -6.6757576357e-01, 1.1751096555e-01, -1.2677815461e-02,
            7.9689343489e-04)


def _fc_poly(u):
    acc = np.float32(_FC_COEF[6])
    for c in _FC_COEF[5::-1]:
        acc = acc * u + np.float32(c)
    return jnp.where(u <= 1.0, acc, 0.0)


def _aev_body(species_ref, coords_ref, posj_ref, posk_ref, pidx_ref,
              dgj_ref, dgk_ref, posr_ref, shfr_ref, dgr_ref, out_ref):
    pi = np.float32(np.pi)

    sp = species_ref[:, 0, :]
    pos = coords_ref[:, :, :]

    posr = posr_ref[:, :, :]
    shfr = shfr_ref[0, :, :]
    dgr = dgr_ref[0, :, :]

    djr = pos[:, :, :, None] - posr[:, :, None, :]
    d2_r = jnp.sum(djr * djr, axis=1)
    fc_rf = _fc_poly(d2_r * np.float32(1.0 / (_RCR * _RCR)))
    fc_rf = fc_rf * (0.25 * (1.0 - dgr))
    d_r = jnp.sqrt(d2_r + dgr)
    rad_f = jnp.exp(-_ETA_R * (d_r - shfr) ** 2) * fc_rf
    sidx = jax.lax.broadcasted_iota(jnp.int32, (_MB, _A, _NUM_SPECIES), 2)
    oh = (sp[:, :, None] == sidx).astype(jnp.float32)
    rad_sf = jax.lax.dot_general(oh, rad_f, (((1,), (1,)), ((0,), (0,))),
                                 preferred_element_type=jnp.float32)
    radial = jnp.transpose(rad_sf.reshape(_MB, _NUM_SPECIES, _A, 16),
                           (0, 2, 1, 3)).reshape(_MB, _A, _RADIAL_F)

    posj = posj_ref[:, :, :]
    posk = posk_ref[:, :, :]
    diag_ij = dgj_ref[0, :, :][None]
    diag_ik = dgk_ref[0, :, :][None]

    dj = pos[:, :, :, None] - posj[:, :, None, :]
    d2_ij = jnp.sum(dj * dj, axis=1)
    dk = pos[:, :, :, None] - posk[:, :, None, :]
    d2_ik = jnp.sum(dk * dk, axis=1)
    ejk = posj - posk
    e2 = ejk * ejk
    d2_jk = (e2[:, 0:1, :] + e2[:, 1:2, :] + e2[:, 2:3, :])

    s2_ij = d2_ij + diag_ij
    s2_ik = d2_ik + diag_ik
    inv_ij = jax.lax.rsqrt(s2_ij)
    inv_ik = jax.lax.rsqrt(s2_ik)
    d_ij = s2_ij * inv_ij
    d_ik = s2_ik * inv_ik

    rca2inv = np.float32(1.0 / (_RCA * _RCA))
    fc_ij = _fc_poly(d2_ij * rca2inv) * (1.0 - diag_ij)
    fc_ik = _fc_poly(d2_ik * rca2inv) * (1.0 - diag_ik)
    w = 2.0 * fc_ij * fc_ik

    dotv = 0.5 * (d2_ij + d2_ik - d2_jk)
    cth = 0.95 * dotv * inv_ij * inv_ik
    sth = jnp.sqrt(jnp.maximum(1.0 - cth * cth, 0.0))
    dsum = jnp.minimum(0.5 * (d_ij + d_ik), 4.0)

    f2_0 = jnp.exp(-_ETA_A * (dsum - 0.9) ** 2)
    r = jnp.exp(10.4 * dsum)
    f2_1 = f2_0 * (r * np.float32(np.exp(-10.4 * 0.9 - 3.38)))
    f2_2 = f2_1 * (r * np.float32(np.exp(-10.4 * 1.55 - 3.38)))
    f2_3 = f2_2 * (r * np.float32(np.exp(-10.4 * 2.2 - 3.38)))

    shfz = ((pi / 16.0) + (pi / 8.0)
            * jax.lax.broadcasted_iota(jnp.int32, (1, 1, 8, 1), 2)
            .astype(jnp.float32))
    czh = 0.5 * jnp.cos(shfz)
    szh = 0.5 * jnp.sin(shfz)
    base = 0.5 + cth[:, :, None, :] * czh + sth[:, :, None, :] * szh
    f1 = base * base
    f1 = f1 * f1
    f1 = f1 * f1
    f1 = f1 * f1
    f1 = f1 * f1

    ang = jnp.concatenate(
        [(w * f2_0)[:, :, None, :] * f1, (w * f2_1)[:, :, None, :] * f1,
         (w * f2_2)[:, :, None, :] * f1, (w * f2_3)[:, :, None, :] * f1],
        axis=2)
    ang = ang.reshape(_MB, _A * 32, _NQ)

    pidx = jnp.broadcast_to(pidx_ref[:, :, :], (_MB, _NUM_PAIRS, _NQ))
    pslot = jax.lax.broadcasted_iota(jnp.int32, (_MB, _NUM_PAIRS, _NQ), 1)
    p_oht = (pidx == pslot).astype(jnp.float32)

    ang_p = jax.lax.dot_general(p_oht, ang, (((2,), (2,)), ((0,), (0,))),
                                preferred_element_type=jnp.float32)
    angular = jnp.transpose(ang_p.reshape(_MB, _NUM_PAIRS, _A, 32),
                            (0, 2, 1, 3)).reshape(_MB, _A, _ANGULAR_F)

    out_ref[:, :, :] = jnp.concatenate([radial, angular], axis=2)


@jax.jit
def _aev_pallas(species, coordinates):
    M, A = species.shape
    sp3 = species.astype(jnp.int32).reshape(M, 1, A)
    coords_t = jnp.transpose(coordinates, (0, 2, 1))

    jq = jnp.asarray(_JQ, dtype=jnp.int32)
    kq = jnp.asarray(_KQ, dtype=jnp.int32)
    npad = _NQ - _NPAIR
    posj = jnp.concatenate(
        [jnp.take(coords_t, jq, axis=2),
         jnp.full((M, 3, npad), 1.0e4, jnp.float32)], axis=2)
    posk = jnp.concatenate(
        [jnp.take(coords_t, kq, axis=2),
         jnp.full((M, 3, npad), 2.0e4, jnp.float32)], axis=2)

    spi = species.astype(jnp.int32)
    spj = jnp.take(spi, jq, axis=1)
    spk = jnp.take(spi, kq, axis=1)
    mn = jnp.minimum(spj, spk)
    mx = jnp.maximum(spj, spk)
    pidx = (mn * (7 - mn)) // 2 + mx
    pidx = jnp.pad(pidx, ((0, 0), (0, npad))).reshape(M, 1, _NQ)

    jq_pad = np.pad(_JQ, (0, npad), constant_values=-1)
    kq_pad = np.pad(_KQ, (0, npad), constant_values=-1)
    irows = np.arange(A)[:, None]
    dgj = jnp.asarray((jq_pad[None, :] == irows).astype(np.float32)
                      ).reshape(1, A, _NQ)
    dgk = jnp.asarray((kq_pad[None, :] == irows).astype(np.float32)
                      ).reshape(1, A, _NQ)

    posr = jnp.repeat(coords_t, 16, axis=2)
    shfr_np = np.tile(0.9 + 0.26875 * np.arange(16, dtype=np.float32), A)
    shfr_flat = jnp.asarray(shfr_np).reshape(1, 1, _NL)
    dgr = jnp.asarray(
        ((np.arange(_NL) // 16)[None, :] == irows).astype(np.float32)
    ).reshape(1, A, _NL)

    out = pl.pallas_call(
        _aev_body,
        grid=(M // _MB,),
        in_specs=[
            pl.BlockSpec((_MB, 1, A), lambda m: (m, 0, 0)),
            pl.BlockSpec((_MB, 3, A), lambda m: (m, 0, 0)),
            pl.BlockSpec((_MB, 3, _NQ), lambda m: (m, 0, 0)),
            pl.BlockSpec((_MB, 3, _NQ), lambda m: (m, 0, 0)),
            pl.BlockSpec((_MB, 1, _NQ), lambda m: (m, 0, 0)),
            pl.BlockSpec((1, A, _NQ), lambda m: (0, 0, 0)),
            pl.BlockSpec((1, A, _NQ), lambda m: (0, 0, 0)),
            pl.BlockSpec((_MB, 3, _NL), lambda m: (m, 0, 0)),
            pl.BlockSpec((1, 1, _NL), lambda m: (0, 0, 0)),
            pl.BlockSpec((1, A, _NL), lambda m: (0, 0, 0)),
        ],
        out_specs=pl.BlockSpec((_MB, A, _RADIAL_F + _ANGULAR_F),
                               lambda m: (m, 0, 0)),
        out_shape=jax.ShapeDtypeStruct((M, A, _RADIAL_F + _ANGULAR_F),
                                       jnp.float32),
    )(sp3, coords_t, posj, posk, pidx, dgj, dgk, posr, shfr_flat, dgr)
    return out


def kernel(species, coordinates):
    aev = _aev_pallas(species, coordinates)
    return (species, aev)


# speedup vs baseline: 1.0272x; 1.0007x over previous
import jax
import jax.numpy as jnp
import numpy as np
from jax.experimental import pallas as pl

_RCR = 5.2
_RCA = 3.5
_NUM_SPECIES = 4
_NUM_PAIRS = 10
_ETA_R = 16.0
_ETA_A = 8.0
_A = 32
_NQ = 512
_NPAIR = _A * (_A - 1) // 2
_NL = _A * 16
_MB = 8
_RADIAL_F = _NUM_SPECIES * 16
_ANGULAR_F = _NUM_PAIRS * 4 * 8

_JQ, _KQ = np.triu_indices(_A, k=1)

_FC_COEF = (9.9999998695e-01, -2.4674003665e+00, 2.0293461123e+00,
            -6.6757576357e-01, 1.1751096555e-01, -1.2677815461e-02,
            7.9689343489e-04)


def _fc_poly(u):
    acc = np.float32(_FC_COEF[6])
    for c in _FC_COEF[5::-1]:
        acc = acc * u + np.float32(c)
    return jnp.where(u <= 1.0, acc, 0.0)


def _aev_body(species_ref, coords_ref, posj_ref, posk_ref, pidx_ref,
              dgj_ref, dgk_ref, posr_ref, shfr_ref, dgr_ref, out_ref):
    pi = np.float32(np.pi)

    sp = species_ref[:, 0, :]
    pos = coords_ref[:, :, :]

    posr = posr_ref[:, :, :]
    shfr = shfr_ref[0, :, :]
    dgr = dgr_ref[0, :, :]

    djr = pos[:, :, :, None] - posr[:, :, None, :]
    d2_r = jnp.sum(djr * djr, axis=1)
    fc_rf = _fc_poly(d2_r * np.float32(1.0 / (_RCR * _RCR)))
    fc_rf = fc_rf * (0.25 * (1.0 - dgr))
    d_r = jnp.sqrt(d2_r + dgr)
    rad_f = jnp.exp(-_ETA_R * (d_r - shfr) ** 2) * fc_rf
    sidx = jax.lax.broadcasted_iota(jnp.int32, (_MB, _A, _NUM_SPECIES), 2)
    oh = (sp[:, :, None] == sidx).astype(jnp.float32)
    rad_sf = jax.lax.dot_general(oh, rad_f, (((1,), (1,)), ((0,), (0,))),
                                 preferred_element_type=jnp.float32)
    radial = jnp.transpose(rad_sf.reshape(_MB, _NUM_SPECIES, _A, 16),
                           (0, 2, 1, 3)).reshape(_MB, _A, _RADIAL_F)

    posj = posj_ref[:, :, :]
    posk = posk_ref[:, :, :]
    diag_ij = dgj_ref[0, :, :][None]
    diag_ik = dgk_ref[0, :, :][None]

    dj = pos[:, :, :, None] - posj[:, :, None, :]
    d2_ij = jnp.sum(dj * dj, axis=1)
    dk = pos[:, :, :, None] - posk[:, :, None, :]
    d2_ik = jnp.sum(dk * dk, axis=1)
    ejk = posj - posk
    e2 = ejk * ejk
    d2_jk = (e2[:, 0:1, :] + e2[:, 1:2, :] + e2[:, 2:3, :])

    s2_ij = d2_ij + diag_ij
    s2_ik = d2_ik + diag_ik
    inv_ij = jax.lax.rsqrt(s2_ij)
    inv_ik = jax.lax.rsqrt(s2_ik)
    d_ij = s2_ij * inv_ij
    d_ik = s2_ik * inv_ik

    rca2inv = np.float32(1.0 / (_RCA * _RCA))
    fc_ij = _fc_poly(d2_ij * rca2inv) * (1.0 - diag_ij)
    fc_ik = _fc_poly(d2_ik * rca2inv) * (1.0 - diag_ik)
    w = 2.0 * fc_ij * fc_ik

    dotv = 0.5 * (d2_ij + d2_ik - d2_jk)
    cth = 0.95 * dotv * inv_ij * inv_ik
    sth = jnp.sqrt(jnp.maximum(1.0 - cth * cth, 0.0))
    dsum = jnp.minimum(0.5 * (d_ij + d_ik), 4.0)

    f2_0 = jnp.exp(-_ETA_A * (dsum - 0.9) ** 2)
    r = jnp.exp(10.4 * dsum)
    f2_1 = f2_0 * (r * np.float32(np.exp(-10.4 * 0.9 - 3.38)))
    f2_2 = f2_1 * (r * np.float32(np.exp(-10.4 * 1.55 - 3.38)))
    f2_3 = f2_2 * (r * np.float32(np.exp(-10.4 * 2.2 - 3.38)))

    shfz = ((pi / 16.0) + (pi / 8.0)
            * jax.lax.broadcasted_iota(jnp.int32, (1, 1, 8, 1), 2)
            .astype(jnp.float32))
    czh = 0.5 * jnp.cos(shfz)
    szh = 0.5 * jnp.sin(shfz)
    base = 0.5 + cth[:, :, None, :] * czh + sth[:, :, None, :] * szh
    f1 = base * base
    f1 = f1 * f1
    f1 = f1 * f1
    f1 = f1 * f1
    f1 = f1 * f1

    ang = jnp.concatenate(
        [(w * f2_0)[:, :, None, :] * f1, (w * f2_1)[:, :, None, :] * f1,
         (w * f2_2)[:, :, None, :] * f1, (w * f2_3)[:, :, None, :] * f1],
        axis=2)
    ang = ang.reshape(_MB, _A * 32, _NQ)

    pidx = jnp.broadcast_to(pidx_ref[:, :, :], (_MB, _NUM_PAIRS, _NQ))
    pslot = jax.lax.broadcasted_iota(jnp.int32, (_MB, _NUM_PAIRS, _NQ), 1)
    p_oht = (pidx == pslot).astype(jnp.float32)

    ang_p = jax.lax.dot_general(p_oht, ang, (((2,), (2,)), ((0,), (0,))),
                                preferred_element_type=jnp.float32)
    angular = jnp.transpose(ang_p.reshape(_MB, _NUM_PAIRS, _A, 32),
                            (0, 2, 1, 3)).reshape(_MB, _A, _ANGULAR_F)

    out_ref[:, :, :] = jnp.concatenate([radial, angular], axis=2)


@jax.jit
def _aev_pallas(species, coordinates):
    M, A = species.shape
    sp3 = species.astype(jnp.int32).reshape(M, 1, A)
    coords_t = jnp.transpose(coordinates, (0, 2, 1))

    jq = jnp.asarray(_JQ, dtype=jnp.int32)
    kq = jnp.asarray(_KQ, dtype=jnp.int32)
    npad = _NQ - _NPAIR
    posj = jnp.concatenate(
        [jnp.take(coords_t, jq, axis=2),
         jnp.full((M, 3, npad), 1.0e4, jnp.float32)], axis=2)
    posk = jnp.concatenate(
        [jnp.take(coords_t, kq, axis=2),
         jnp.full((M, 3, npad), 2.0e4, jnp.float32)], axis=2)

    spi = species.astype(jnp.int32)
    spj = jnp.take(spi, jq, axis=1)
    spk = jnp.take(spi, kq, axis=1)
    mn = jnp.minimum(spj, spk)
    mx = jnp.maximum(spj, spk)
    pidx = (mn * (7 - mn)) // 2 + mx
    pidx = jnp.pad(pidx, ((0, 0), (0, npad))).reshape(M, 1, _NQ)

    jq_pad = np.pad(_JQ, (0, npad), constant_values=-1)
    kq_pad = np.pad(_KQ, (0, npad), constant_values=-1)
    irows = np.arange(A)[:, None]
    dgj = jnp.asarray((jq_pad[None, :] == irows).astype(np.float32)
                      ).reshape(1, A, _NQ)
    dgk = jnp.asarray((kq_pad[None, :] == irows).astype(np.float32)
                      ).reshape(1, A, _NQ)

    posr = jnp.repeat(coords_t, 16, axis=2)
    shfr_np = np.tile(0.9 + 0.26875 * np.arange(16, dtype=np.float32), A)
    shfr_flat = jnp.asarray(shfr_np).reshape(1, 1, _NL)
    dgr = jnp.asarray(
        ((np.arange(_NL) // 16)[None, :] == irows).astype(np.float32)
    ).reshape(1, A, _NL)

    out = pl.pallas_call(
        _aev_body,
        grid=(M // _MB,),
        in_specs=[
            pl.BlockSpec((_MB, 1, A), lambda m: (m, 0, 0)),
            pl.BlockSpec((_MB, 3, A), lambda m: (m, 0, 0)),
            pl.BlockSpec((_MB, 3, _NQ), lambda m: (m, 0, 0)),
            pl.BlockSpec((_MB, 3, _NQ), lambda m: (m, 0, 0)),
            pl.BlockSpec((_MB, 1, _NQ), lambda m: (m, 0, 0)),
            pl.BlockSpec((1, A, _NQ), lambda m: (0, 0, 0)),
            pl.BlockSpec((1, A, _NQ), lambda m: (0, 0, 0)),
            pl.BlockSpec((_MB, 3, _NL), lambda m: (m, 0, 0)),
            pl.BlockSpec((1, 1, _NL), lambda m: (0, 0, 0)),
            pl.BlockSpec((1, A, _NL), lambda m: (0, 0, 0)),
        ],
        out_specs=pl.BlockSpec((_MB, A, _RADIAL_F + _ANGULAR_F),
                               lambda m: (m, 0, 0)),
        out_shape=jax.ShapeDtypeStruct((M, A, _RADIAL_F + _ANGULAR_F),
                                       jnp.float32),
    )(sp3, coords_t, posj, posk, pidx, dgj, dgk, posr, shfr_flat, dgr)
    return out


def kernel(species, coordinates):
    aev = _aev_pallas(species, coordinates)
    return (species, aev)
